# Initial kernel scaffold; baseline (speedup 1.0000x reference)
import jax
import jax.numpy as jnp
from jax.experimental import pallas as pl


def kernel(features, cluster_features, edge_index, raw_affine, affine, conv_w0, conv_b0, conv_w1, conv_b1, conv_w2, conv_b2, conv_w3, conv_b3, src_w, src_b, dst_w, dst_b, prelu1, cls_w1, cls_b1, prelu2, cls_w2, cls_b2):
    raise NotImplementedError("write your pallas kernel here")


# speedup vs baseline: 3.6412x; 3.6412x over previous
import functools

import jax
import jax.numpy as jnp
from jax import lax
from jax.experimental import pallas as pl
from jax.experimental.pallas import tpu as pltpu
from jax.experimental.pallas import tpu_sc as plsc

_N = 10000
_E = 320000
_CH = 80
_NCHUNKS = _E // _CH
_NC, _NS, _L = 2, 16, 16
_RPT = 624
_REM = _N - _NS * _RPT
_SB = 25


def _mxdot(x, w):
    return jnp.dot(x.astype(jnp.bfloat16), w.astype(jnp.bfloat16),
                   preferred_element_type=jnp.float32)


def _sc_mesh():
    return plsc.VectorSubcoreMesh(
        core_axis_name="c", subcore_axis_name="s",
        num_cores=_NC, num_subcores=_NS)


def _segsum_body(split_channels, nch_pt, x_hbm, src_hbm, dst_hbm, aff_hbm,
                 zeros_hbm, out_hbm, srcb, dstb, affb, idxb, rowbuf, acc, sem):
    c = lax.axis_index("c")
    s = lax.axis_index("s")
    pltpu.sync_copy(zeros_hbm.at[pl.ds(s * _RPT, _RPT)],
                    acc.at[pl.ds(s * _RPT, _RPT)])

    @pl.when(s == _NS - 1)
    def _zero_tail():
        pltpu.sync_copy(zeros_hbm.at[pl.ds(_NS * _RPT, _REM)],
                        acc.at[pl.ds(_NS * _RPT, _REM)])

    if split_channels:
        tid = s
        goff = c * _N
    else:
        tid = c * _NS + s
        goff = c * 0
    plsc.subcore_barrier()

    def block(bk, bc):
        pltpu.sync_copy(src_hbm.at[tid, bk], srcb)
        pltpu.sync_copy(dst_hbm.at[tid, bk], dstb)
        pltpu.sync_copy(aff_hbm.at[tid, bk], affb)

        def chunk(j, carry):
            for k in range(_CH // _L):
                idxb[pl.ds(k * _L, _L)] = srcb[j, pl.ds(k * _L, _L)] + goff
            pltpu.async_copy(x_hbm.at[idxb], rowbuf, sem).wait()

            def group(g, gc):
                av = affb[j, pl.ds(g * _L, _L)]
                for r in range(_L):
                    a = av[r]
                    rb = g * _L + r
                    for k in range(128 // _L):
                        sl = pl.ds(k * _L, _L)
                        rowbuf[rb, sl] = rowbuf[rb, sl] * a
                return gc
            lax.fori_loop(0, _CH // _L, group, 0)
            pltpu.sync_copy(rowbuf, acc.at[dstb.at[j]], add=True)
            return carry
        lax.fori_loop(0, _SB, chunk, 0)
        return bc
    lax.fori_loop(0, nch_pt // _SB, block, 0)
    plsc.subcore_barrier()
    pltpu.sync_copy(acc.at[pl.ds(s * _RPT, _RPT)],
                    out_hbm.at[pl.ds(c * _N + s * _RPT, _RPT)])

    @pl.when(s == _NS - 1)
    def _copy_tail():
        pltpu.sync_copy(acc.at[pl.ds(_NS * _RPT, _REM)],
                        out_hbm.at[pl.ds(c * _N + _NS * _RPT, _REM)])


def _sc_segsum(x_flat, src3, dst3, aff3, zeros, split_channels):
    nch_pt = _NCHUNKS // _NS if split_channels else _NCHUNKS // (_NS * _NC)
    body = functools.partial(_segsum_body, split_channels, nch_pt)
    f = pl.kernel(
        body,
        out_type=jax.ShapeDtypeStruct((2 * _N, 128), jnp.float32),
        mesh=_sc_mesh(),
        scratch_types=[
            pltpu.VMEM((_SB, _CH), jnp.int32),
            pltpu.VMEM((_SB, _CH), jnp.int32),
            pltpu.VMEM((_SB, _CH), jnp.float32),
            pltpu.VMEM((_CH,), jnp.int32),
            pltpu.VMEM((_CH, 128), jnp.float32),
            pltpu.VMEM_SHARED((_N, 128), jnp.float32),
            pltpu.SemaphoreType.DMA,
        ],
    )
    return f(x_flat, src3, dst3, aff3, zeros)


def _head_body(sd_hbm, src_hbm, dst_hbm, out_hbm, srcb, dstb, idxb,
               abuf, bbuf, sem):
    c = lax.axis_index("c")
    s = lax.axis_index("s")
    npt = _NCHUNKS // (_NS * _NC)
    wid = c * _NS + s
    base = wid * npt
    pltpu.sync_copy(src_hbm.at[wid], srcb)
    pltpu.sync_copy(dst_hbm.at[wid], dstb)

    def chunk(j, carry):
        pltpu.async_copy(sd_hbm.at[srcb.at[j]], abuf, sem).wait()
        for k in range(_CH // _L):
            idxb[pl.ds(k * _L, _L)] = dstb[j, pl.ds(k * _L, _L)] + _N
        pltpu.async_copy(sd_hbm.at[idxb], bbuf, sem).wait()

        def row(r, rc):
            for k in range(128 // _L):
                sl = pl.ds(k * _L, _L)
                abuf[r, sl] = abuf[r, sl] + bbuf[r, sl]
            return rc
        lax.fori_loop(0, _CH, row, 0)
        pltpu.sync_copy(abuf, out_hbm.at[pl.ds((base + j) * _CH, _CH)])
        return carry
    lax.fori_loop(0, npt, chunk, 0)


def _sc_head(sd_flat, src3, dst3):
    f = pl.kernel(
        _head_body,
        out_type=jax.ShapeDtypeStruct((_E, 128), jnp.float32),
        mesh=_sc_mesh(),
        scratch_types=[
            pltpu.VMEM((_NCHUNKS // (_NS * _NC), _CH), jnp.int32),
            pltpu.VMEM((_NCHUNKS // (_NS * _NC), _CH), jnp.int32),
            pltpu.VMEM((_CH,), jnp.int32),
            pltpu.VMEM((_CH, 128), jnp.float32),
            pltpu.VMEM((_CH, 128), jnp.float32),
            pltpu.SemaphoreType.DMA,
        ],
    )
    return f(sd_flat, src3, dst3)


def _den_body(m_hbm, dst_hbm, zeros_hbm, out_hbm, dstb, mb, rowbuf, acc):
    c = lax.axis_index("c")
    s = lax.axis_index("s")
    wid = c * _NS + s
    nblk = _NCHUNKS // (_NS * _NC) // _SB
    pltpu.sync_copy(zeros_hbm.at[pl.ds(s * _RPT, _RPT)],
                    acc.at[pl.ds(s * _RPT, _RPT)])

    @pl.when(s == _NS - 1)
    def _zero_tail():
        pltpu.sync_copy(zeros_hbm.at[pl.ds(_NS * _RPT, _REM)],
                        acc.at[pl.ds(_NS * _RPT, _REM)])

    plsc.subcore_barrier()
    lanes = lax.broadcasted_iota(jnp.int32, (_L,), 0)
    mask0 = lanes == 0
    deg1 = jnp.where(lanes == 1, 1.0, 0.0).astype(jnp.float32)
    zf = jnp.zeros((_L,), jnp.float32)

    def zrow(i, zc):
        for k in range(128 // _L):
            rowbuf[i, pl.ds(k * _L, _L)] = zf
        return zc
    lax.fori_loop(0, _CH, zrow, 0)

    def block(bk, bc):
        pltpu.sync_copy(dst_hbm.at[wid, bk], dstb)
        pltpu.sync_copy(m_hbm.at[wid, bk], mb)

        def chunk(j, carry):
            def group(g, gc):
                mv = mb[j, pl.ds(g * _L, _L)]
                for r in range(_L):
                    rowbuf[g * _L + r, pl.ds(0, _L)] = jnp.where(
                        mask0, mv[r], deg1)
                return gc
            lax.fori_loop(0, _CH // _L, group, 0)
            pltpu.sync_copy(rowbuf, acc.at[dstb.at[j]], add=True)
            return carry
        lax.fori_loop(0, _SB, chunk, 0)
        return bc
    lax.fori_loop(0, nblk, block, 0)
    plsc.subcore_barrier()
    pltpu.sync_copy(acc.at[pl.ds(s * _RPT, _RPT)],
                    out_hbm.at[pl.ds(c * _N + s * _RPT, _RPT)])

    @pl.when(s == _NS - 1)
    def _copy_tail():
        pltpu.sync_copy(acc.at[pl.ds(_NS * _RPT, _REM)],
                        out_hbm.at[pl.ds(c * _N + _NS * _RPT, _REM)])


def _sc_den_partials(m4, dst4, zeros):
    f = pl.kernel(
        _den_body,
        out_type=jax.ShapeDtypeStruct((2 * _N, 128), jnp.float32),
        mesh=_sc_mesh(),
        scratch_types=[
            pltpu.VMEM((_SB, _CH), jnp.int32),
            pltpu.VMEM((_SB, _CH), jnp.float32),
            pltpu.VMEM((_CH, 128), jnp.float32),
            pltpu.VMEM_SHARED((_N, 128), jnp.float32),
        ],
    )
    return f(m4, dst4, zeros)


_BN = 400
_BE = 2000


def _tc_conv(x2, agg2, w, b, cout):
    def body(x_ref, a_ref, w_ref, b_ref, out_ref):
        h = jnp.concatenate([x_ref[0], x_ref[1], a_ref[0], a_ref[1]], axis=1)
        y = _mxdot(h, w_ref[...])
        y = jnp.maximum(y + b_ref[...], 0.0)
        if cout == 256:
            out_ref[0] = y[:, 0:128]
            out_ref[1] = y[:, 128:256]
        else:
            out_ref[...] = y

    if cout == 256:
        out_shape = jax.ShapeDtypeStruct((2, _N, 128), jnp.float32)
        out_spec = pl.BlockSpec((2, _BN, 128), lambda i: (0, i, 0))
    else:
        out_shape = jax.ShapeDtypeStruct((_N, 128), jnp.float32)
        out_spec = pl.BlockSpec((_BN, 128), lambda i: (i, 0))
    return pl.pallas_call(
        body,
        grid=(_N // _BN,),
        in_specs=[
            pl.BlockSpec((2, _BN, 128), lambda i: (0, i, 0)),
            pl.BlockSpec((2, _BN, 128), lambda i: (0, i, 0)),
            pl.BlockSpec((512, cout), lambda i: (0, 0)),
            pl.BlockSpec((cout,), lambda i: (0,)),
        ],
        out_specs=out_spec,
        out_shape=out_shape,
    )(x2, agg2, w, b)


def _tc_conv3(x3, aggp, w, b, src_w, src_b, dst_w, dst_b):
    def body(x_ref, a_ref, w_ref, b_ref, sw_ref, sb_ref, dw_ref, db_ref,
             out_ref):
        agg = a_ref[0] + a_ref[1]
        h = jnp.concatenate([x_ref[...], agg], axis=1)
        y = _mxdot(h, w_ref[...])
        y = jnp.maximum(y + b_ref[...], 0.0)
        out_ref[0] = _mxdot(y, sw_ref[...]) + sb_ref[...]
        out_ref[1] = _mxdot(y, dw_ref[...]) + db_ref[...]

    return pl.pallas_call(
        body,
        grid=(_N // _BN,),
        in_specs=[
            pl.BlockSpec((_BN, 128), lambda i: (i, 0)),
            pl.BlockSpec((2, _BN, 128), lambda i: (0, i, 0)),
            pl.BlockSpec((256, 128), lambda i: (0, 0)),
            pl.BlockSpec((128,), lambda i: (0,)),
            pl.BlockSpec((128, 128), lambda i: (0, 0)),
            pl.BlockSpec((128,), lambda i: (0,)),
            pl.BlockSpec((128, 128), lambda i: (0, 0)),
            pl.BlockSpec((128,), lambda i: (0,)),
        ],
        out_specs=pl.BlockSpec((2, _BN, 128), lambda i: (0, i, 0)),
        out_shape=jax.ShapeDtypeStruct((2, _N, 128), jnp.float32),
    )(x3, aggp, w, b, src_w, src_b, dst_w, dst_b)


def _tc_head(hsum, raw2, p1, w1, b1, p2, w2, b2):
    def body(h_ref, r_ref, p1_ref, w1_ref, b1_ref, p2_ref, w2_ref, b2_ref,
             pc_ref, m_ref):
        h0 = h_ref[...]
        h = jnp.where(h0 > 0, h0, h0 * p1_ref[...])
        h = _mxdot(h, w1_ref[...]) + b1_ref[...]
        h = jnp.where(h > 0, h, h * p2_ref[...])
        pc = _mxdot(h, w2_ref[...]) + b2_ref[...]
        pc_ref[...] = pc
        md = jnp.tanh((pc[:, 1] - pc[:, 0]) * 0.5)
        m_ref[0, 0, :] = r_ref[0, 0, :] * md

    return pl.pallas_call(
        body,
        grid=(_E // _BE,),
        in_specs=[
            pl.BlockSpec((_BE, 128), lambda i: (i, 0)),
            pl.BlockSpec((1, 1, _BE), lambda i: (i, 0, 0)),
            pl.BlockSpec((128,), lambda i: (0,)),
            pl.BlockSpec((128, 128), lambda i: (0, 0)),
            pl.BlockSpec((128,), lambda i: (0,)),
            pl.BlockSpec((128,), lambda i: (0,)),
            pl.BlockSpec((128, 2), lambda i: (0, 0)),
            pl.BlockSpec((2,), lambda i: (0,)),
        ],
        out_specs=[
            pl.BlockSpec((_BE, 2), lambda i: (i, 0)),
            pl.BlockSpec((1, 1, _BE), lambda i: (i, 0, 0)),
        ],
        out_shape=[
            jax.ShapeDtypeStruct((_E, 2), jnp.float32),
            jax.ShapeDtypeStruct((_E // _BE, 1, _BE), jnp.float32),
        ],
    )(hsum, raw2, p1, w1, b1, p2, w2, b2)


def _tc_den(parts):
    def body(p_ref, out_ref):
        ssum = p_ref[0, :, 0] + p_ref[1, :, 0]
        deg = p_ref[0, :, 1] + p_ref[1, :, 1]
        out_ref[0, :] = ssum / jnp.maximum(deg, 1.0)

    return pl.pallas_call(
        body,
        out_shape=jax.ShapeDtypeStruct((1, _N), jnp.float32),
    )(parts)


def kernel(features, cluster_features, edge_index, raw_affine, affine,
           conv_w0, conv_b0, conv_w1, conv_b1, conv_w2, conv_b2, conv_w3,
           conv_b3, src_w, src_b, dst_w, dst_b, prelu1, cls_w1, cls_b1,
           prelu2, cls_w2, cls_b2):
    n16 = _NCHUNKS // _NS
    n32 = _NCHUNKS // (_NS * _NC)
    src16 = edge_index[0].reshape(_NS, n16 // _SB, _SB, _CH)
    dst16 = edge_index[1].reshape(_NS, n16 // _SB, _SB, _CH)
    aff16 = affine.reshape(_NS, n16 // _SB, _SB, _CH)
    src32b = edge_index[0].reshape(_NS * _NC, n32 // _SB, _SB, _CH)
    dst32b = edge_index[1].reshape(_NS * _NC, n32 // _SB, _SB, _CH)
    aff32b = affine.reshape(_NS * _NC, n32 // _SB, _SB, _CH)
    src32 = edge_index[0].reshape(_NS * _NC, n32, _CH)
    dst32 = edge_index[1].reshape(_NS * _NC, n32, _CH)
    zeros = jnp.zeros((_N, 128), jnp.float32)

    x2 = jnp.stack([features, cluster_features])
    for w, b, cout in ((conv_w0, conv_b0, 256), (conv_w1, conv_b1, 256),
                       (conv_w2, conv_b2, 128)):
        agg = _sc_segsum(x2.reshape(2 * _N, 128), src16, dst16, aff16, zeros,
                         split_channels=True)
        out = _tc_conv(x2, agg.reshape(2, _N, 128), w, b, cout)
        x2 = out
    x3 = x2
    aggp = _sc_segsum(x3, src32b, dst32b, aff32b, zeros, split_channels=False)
    sd = _tc_conv3(x3, aggp.reshape(2, _N, 128), conv_w3, conv_b3,
                   src_w, src_b, dst_w, dst_b)
    hsum = _sc_head(sd.reshape(2 * _N, 128), src32, dst32)
    pred_conn, m2 = _tc_head(hsum, raw_affine.reshape(_E // _BE, 1, _BE),
                             prelu1, cls_w1, cls_b1, prelu2, cls_w2, cls_b2)
    parts = _sc_den_partials(m2.reshape(_NS * _NC, n32 // _SB, _SB, _CH),
                             dst32b, zeros)
    pred_den = _tc_den(parts.reshape(2, _N, 128)).reshape(_N)
    return pred_conn, pred_den


# speedup vs baseline: 4.5956x; 1.2621x over previous
import functools

import jax
import jax.numpy as jnp
from jax import lax
from jax.experimental import pallas as pl
from jax.experimental.pallas import tpu as pltpu
from jax.experimental.pallas import tpu_sc as plsc

_N = 10000
_E = 320000
_CH = 80
_NCHUNKS = _E // _CH
_NC, _NS, _L = 2, 16, 16
_RPT = 624
_REM = _N - _NS * _RPT
_SB = 25


def _mxdot(x, w):
    return jnp.dot(x.astype(jnp.bfloat16), w.astype(jnp.bfloat16),
                   preferred_element_type=jnp.float32)


def _sc_mesh():
    return plsc.VectorSubcoreMesh(
        core_axis_name="c", subcore_axis_name="s",
        num_cores=_NC, num_subcores=_NS)


def _segsum_body(split_channels, nch_pt, x_hbm, src_hbm, dst_hbm, aff_hbm,
                 zeros_hbm, out_hbm, srcb, dstb, affb, idxb0, idxb1,
                 rowbuf0, rowbuf1, acc, gsem0, gsem1, ssem):
    c = lax.axis_index("c")
    s = lax.axis_index("s")
    pltpu.sync_copy(zeros_hbm.at[pl.ds(s * _RPT, _RPT)],
                    acc.at[pl.ds(s * _RPT, _RPT)])

    @pl.when(s == _NS - 1)
    def _zero_tail():
        pltpu.sync_copy(zeros_hbm.at[pl.ds(_NS * _RPT, _REM)],
                        acc.at[pl.ds(_NS * _RPT, _REM)])

    if split_channels:
        tid = s
        goff = c * _N
    else:
        tid = c * _NS + s
        goff = c * 0
    plsc.subcore_barrier()

    def do_idx(j, idxb):
        for k in range(_CH // _L):
            idxb[pl.ds(k * _L, _L)] = srcb[j, pl.ds(k * _L, _L)] + goff

    def do_scale(j, rowbuf):
        def group(g, gc):
            av = affb[j, pl.ds(g * _L, _L)]
            for r in range(_L):
                a = av[r]
                rb = g * _L + r
                for k in range(128 // _L):
                    sl = pl.ds(k * _L, _L)
                    rowbuf[rb, sl] = rowbuf[rb, sl] * a
            return gc
        lax.fori_loop(0, _CH // _L, group, 0)

    def block(bk, bc):
        pltpu.sync_copy(src_hbm.at[tid, bk], srcb)
        pltpu.sync_copy(dst_hbm.at[tid, bk], dstb)
        pltpu.sync_copy(aff_hbm.at[tid, bk], affb)

        def pair(j2, carry):
            j = 2 * j2
            do_idx(j, idxb0)
            ga = pltpu.make_async_copy(x_hbm.at[idxb0], rowbuf0, gsem0)
            ga.start()
            do_idx(j + 1, idxb1)
            gb = pltpu.make_async_copy(x_hbm.at[idxb1], rowbuf1, gsem1)
            gb.start()
            ga.wait()
            do_scale(j, rowbuf0)
            sa = pltpu.make_async_copy(rowbuf0, acc.at[dstb.at[j]], ssem)
            sa.start(add=True)
            gb.wait()
            do_scale(j + 1, rowbuf1)
            sb = pltpu.make_async_copy(rowbuf1, acc.at[dstb.at[j + 1]], ssem)
            sb.start(add=True)
            sa.wait()
            sb.wait()
            return carry
        lax.fori_loop(0, _SB // 2, pair, 0)
        if _SB % 2:
            jt = _SB - 1
            do_idx(jt, idxb0)
            pltpu.async_copy(x_hbm.at[idxb0], rowbuf0, gsem0).wait()
            do_scale(jt, rowbuf0)
            pltpu.sync_copy(rowbuf0, acc.at[dstb.at[jt]], add=True)
        return bc
    lax.fori_loop(0, nch_pt // _SB, block, 0)
    plsc.subcore_barrier()
    pltpu.sync_copy(acc.at[pl.ds(s * _RPT, _RPT)],
                    out_hbm.at[pl.ds(c * _N + s * _RPT, _RPT)])

    @pl.when(s == _NS - 1)
    def _copy_tail():
        pltpu.sync_copy(acc.at[pl.ds(_NS * _RPT, _REM)],
                        out_hbm.at[pl.ds(c * _N + _NS * _RPT, _REM)])


def _sc_segsum(x_flat, src3, dst3, aff3, zeros, split_channels):
    nch_pt = _NCHUNKS // _NS if split_channels else _NCHUNKS // (_NS * _NC)
    body = functools.partial(_segsum_body, split_channels, nch_pt)
    f = pl.kernel(
        body,
        out_type=jax.ShapeDtypeStruct((2 * _N, 128), jnp.float32),
        mesh=_sc_mesh(),
        scratch_types=[
            pltpu.VMEM((_SB, _CH), jnp.int32),
            pltpu.VMEM((_SB, _CH), jnp.int32),
            pltpu.VMEM((_SB, _CH), jnp.float32),
            pltpu.VMEM((_CH,), jnp.int32),
            pltpu.VMEM((_CH,), jnp.int32),
            pltpu.VMEM((_CH, 128), jnp.float32),
            pltpu.VMEM((_CH, 128), jnp.float32),
            pltpu.VMEM_SHARED((_N, 128), jnp.float32),
            pltpu.SemaphoreType.DMA,
            pltpu.SemaphoreType.DMA,
            pltpu.SemaphoreType.DMA,
        ],
    )
    return f(x_flat, src3, dst3, aff3, zeros)


def _head_body(sd_hbm, src_hbm, dst_hbm, out_hbm, srcb, dstb, idxb,
               abuf, bbuf, sem, sem2):
    c = lax.axis_index("c")
    s = lax.axis_index("s")
    npt = _NCHUNKS // (_NS * _NC)
    wid = c * _NS + s
    base = wid * npt
    pltpu.sync_copy(src_hbm.at[wid], srcb)
    pltpu.sync_copy(dst_hbm.at[wid], dstb)

    def chunk(j, carry):
        ga = pltpu.make_async_copy(sd_hbm.at[srcb.at[j]], abuf, sem)
        ga.start()
        for k in range(_CH // _L):
            idxb[pl.ds(k * _L, _L)] = dstb[j, pl.ds(k * _L, _L)] + _N
        gb = pltpu.make_async_copy(sd_hbm.at[idxb], bbuf, sem2)
        gb.start()
        ga.wait()
        gb.wait()

        def row(r, rc):
            for k in range(128 // _L):
                sl = pl.ds(k * _L, _L)
                abuf[r, sl] = abuf[r, sl] + bbuf[r, sl]
            return rc
        lax.fori_loop(0, _CH, row, 0)
        pltpu.sync_copy(abuf, out_hbm.at[pl.ds((base + j) * _CH, _CH)])
        return carry
    lax.fori_loop(0, npt, chunk, 0)


def _sc_head(sd_flat, src3, dst3):
    f = pl.kernel(
        _head_body,
        out_type=jax.ShapeDtypeStruct((_E, 128), jnp.float32),
        mesh=_sc_mesh(),
        scratch_types=[
            pltpu.VMEM((_NCHUNKS // (_NS * _NC), _CH), jnp.int32),
            pltpu.VMEM((_NCHUNKS // (_NS * _NC), _CH), jnp.int32),
            pltpu.VMEM((_CH,), jnp.int32),
            pltpu.VMEM((_CH, 128), jnp.float32),
            pltpu.VMEM((_CH, 128), jnp.float32),
            pltpu.SemaphoreType.DMA,
            pltpu.SemaphoreType.DMA,
        ],
    )
    return f(sd_flat, src3, dst3)


def _den_body(m_hbm, dst_hbm, zeros_hbm, out_hbm, dstb, mb, rowbuf, acc):
    c = lax.axis_index("c")
    s = lax.axis_index("s")
    wid = c * _NS + s
    nblk = _NCHUNKS // (_NS * _NC) // _SB
    pltpu.sync_copy(zeros_hbm.at[pl.ds(s * _RPT, _RPT)],
                    acc.at[pl.ds(s * _RPT, _RPT)])

    @pl.when(s == _NS - 1)
    def _zero_tail():
        pltpu.sync_copy(zeros_hbm.at[pl.ds(_NS * _RPT, _REM)],
                        acc.at[pl.ds(_NS * _RPT, _REM)])

    plsc.subcore_barrier()
    lanes = lax.broadcasted_iota(jnp.int32, (_L,), 0)
    mask0 = lanes == 0
    deg1 = jnp.where(lanes == 1, 1.0, 0.0).astype(jnp.float32)
    zf = jnp.zeros((_L,), jnp.float32)

    def zrow(i, zc):
        for k in range(128 // _L):
            rowbuf[i, pl.ds(k * _L, _L)] = zf
        return zc
    lax.fori_loop(0, _CH, zrow, 0)

    def block(bk, bc):
        pltpu.sync_copy(dst_hbm.at[wid, bk], dstb)
        pltpu.sync_copy(m_hbm.at[wid, bk], mb)

        def chunk(j, carry):
            def group(g, gc):
                mv = mb[j, pl.ds(g * _L, _L)]
                for r in range(_L):
                    rowbuf[g * _L + r, pl.ds(0, _L)] = jnp.where(
                        mask0, mv[r], deg1)
                return gc
            lax.fori_loop(0, _CH // _L, group, 0)
            pltpu.sync_copy(rowbuf, acc.at[dstb.at[j]], add=True)
            return carry
        lax.fori_loop(0, _SB, chunk, 0)
        return bc
    lax.fori_loop(0, nblk, block, 0)
    plsc.subcore_barrier()
    pltpu.sync_copy(acc.at[pl.ds(s * _RPT, _RPT)],
                    out_hbm.at[pl.ds(c * _N + s * _RPT, _RPT)])

    @pl.when(s == _NS - 1)
    def _copy_tail():
        pltpu.sync_copy(acc.at[pl.ds(_NS * _RPT, _REM)],
                        out_hbm.at[pl.ds(c * _N + _NS * _RPT, _REM)])


def _sc_den_partials(m4, dst4, zeros):
    f = pl.kernel(
        _den_body,
        out_type=jax.ShapeDtypeStruct((2 * _N, 128), jnp.float32),
        mesh=_sc_mesh(),
        scratch_types=[
            pltpu.VMEM((_SB, _CH), jnp.int32),
            pltpu.VMEM((_SB, _CH), jnp.float32),
            pltpu.VMEM((_CH, 128), jnp.float32),
            pltpu.VMEM_SHARED((_N, 128), jnp.float32),
        ],
    )
    return f(m4, dst4, zeros)


_BN = 400
_BE = 2000


def _tc_conv(x2, agg2, w, b, cout):
    def body(x_ref, a_ref, w_ref, b_ref, out_ref):
        h = jnp.concatenate([x_ref[0], x_ref[1], a_ref[0], a_ref[1]], axis=1)
        y = _mxdot(h, w_ref[...])
        y = jnp.maximum(y + b_ref[...], 0.0)
        if cout == 256:
            out_ref[0] = y[:, 0:128]
            out_ref[1] = y[:, 128:256]
        else:
            out_ref[...] = y

    if cout == 256:
        out_shape = jax.ShapeDtypeStruct((2, _N, 128), jnp.float32)
        out_spec = pl.BlockSpec((2, _BN, 128), lambda i: (0, i, 0))
    else:
        out_shape = jax.ShapeDtypeStruct((_N, 128), jnp.float32)
        out_spec = pl.BlockSpec((_BN, 128), lambda i: (i, 0))
    return pl.pallas_call(
        body,
        grid=(_N // _BN,),
        in_specs=[
            pl.BlockSpec((2, _BN, 128), lambda i: (0, i, 0)),
            pl.BlockSpec((2, _BN, 128), lambda i: (0, i, 0)),
            pl.BlockSpec((512, cout), lambda i: (0, 0)),
            pl.BlockSpec((cout,), lambda i: (0,)),
        ],
        out_specs=out_spec,
        out_shape=out_shape,
    )(x2, agg2, w, b)


def _tc_conv3(x3, aggp, w, b, src_w, src_b, dst_w, dst_b):
    def body(x_ref, a_ref, w_ref, b_ref, sw_ref, sb_ref, dw_ref, db_ref,
             out_ref):
        agg = a_ref[0] + a_ref[1]
        h = jnp.concatenate([x_ref[...], agg], axis=1)
        y = _mxdot(h, w_ref[...])
        y = jnp.maximum(y + b_ref[...], 0.0)
        out_ref[0] = _mxdot(y, sw_ref[...]) + sb_ref[...]
        out_ref[1] = _mxdot(y, dw_ref[...]) + db_ref[...]

    return pl.pallas_call(
        body,
        grid=(_N // _BN,),
        in_specs=[
            pl.BlockSpec((_BN, 128), lambda i: (i, 0)),
            pl.BlockSpec((2, _BN, 128), lambda i: (0, i, 0)),
            pl.BlockSpec((256, 128), lambda i: (0, 0)),
            pl.BlockSpec((128,), lambda i: (0,)),
            pl.BlockSpec((128, 128), lambda i: (0, 0)),
            pl.BlockSpec((128,), lambda i: (0,)),
            pl.BlockSpec((128, 128), lambda i: (0, 0)),
            pl.BlockSpec((128,), lambda i: (0,)),
        ],
        out_specs=pl.BlockSpec((2, _BN, 128), lambda i: (0, i, 0)),
        out_shape=jax.ShapeDtypeStruct((2, _N, 128), jnp.float32),
    )(x3, aggp, w, b, src_w, src_b, dst_w, dst_b)


def _tc_head(hsum, raw2, p1, w1, b1, p2, w2, b2):
    def body(h_ref, r_ref, p1_ref, w1_ref, b1_ref, p2_ref, w2_ref, b2_ref,
             pc_ref, m_ref):
        h0 = h_ref[...]
        h = jnp.where(h0 > 0, h0, h0 * p1_ref[...])
        h = _mxdot(h, w1_ref[...]) + b1_ref[...]
        h = jnp.where(h > 0, h, h * p2_ref[...])
        pc = _mxdot(h, w2_ref[...]) + b2_ref[...]
        pc_ref[...] = pc
        md = jnp.tanh((pc[:, 1] - pc[:, 0]) * 0.5)
        m_ref[0, 0, :] = r_ref[0, 0, :] * md

    return pl.pallas_call(
        body,
        grid=(_E // _BE,),
        in_specs=[
            pl.BlockSpec((_BE, 128), lambda i: (i, 0)),
            pl.BlockSpec((1, 1, _BE), lambda i: (i, 0, 0)),
            pl.BlockSpec((128,), lambda i: (0,)),
            pl.BlockSpec((128, 128), lambda i: (0, 0)),
            pl.BlockSpec((128,), lambda i: (0,)),
            pl.BlockSpec((128,), lambda i: (0,)),
            pl.BlockSpec((128, 2), lambda i: (0, 0)),
            pl.BlockSpec((2,), lambda i: (0,)),
        ],
        out_specs=[
            pl.BlockSpec((_BE, 2), lambda i: (i, 0)),
            pl.BlockSpec((1, 1, _BE), lambda i: (i, 0, 0)),
        ],
        out_shape=[
            jax.ShapeDtypeStruct((_E, 2), jnp.float32),
            jax.ShapeDtypeStruct((_E // _BE, 1, _BE), jnp.float32),
        ],
    )(hsum, raw2, p1, w1, b1, p2, w2, b2)


def _tc_den(parts):
    def body(p_ref, out_ref):
        ssum = p_ref[0, :, 0] + p_ref[1, :, 0]
        deg = p_ref[0, :, 1] + p_ref[1, :, 1]
        out_ref[0, :] = ssum / jnp.maximum(deg, 1.0)

    return pl.pallas_call(
        body,
        out_shape=jax.ShapeDtypeStruct((1, _N), jnp.float32),
    )(parts)


def kernel(features, cluster_features, edge_index, raw_affine, affine,
           conv_w0, conv_b0, conv_w1, conv_b1, conv_w2, conv_b2, conv_w3,
           conv_b3, src_w, src_b, dst_w, dst_b, prelu1, cls_w1, cls_b1,
           prelu2, cls_w2, cls_b2):
    n16 = _NCHUNKS // _NS
    n32 = _NCHUNKS // (_NS * _NC)
    src16 = edge_index[0].reshape(_NS, n16 // _SB, _SB, _CH)
    dst16 = edge_index[1].reshape(_NS, n16 // _SB, _SB, _CH)
    aff16 = affine.reshape(_NS, n16 // _SB, _SB, _CH)
    src32b = edge_index[0].reshape(_NS * _NC, n32 // _SB, _SB, _CH)
    dst32b = edge_index[1].reshape(_NS * _NC, n32 // _SB, _SB, _CH)
    aff32b = affine.reshape(_NS * _NC, n32 // _SB, _SB, _CH)
    src32 = edge_index[0].reshape(_NS * _NC, n32, _CH)
    dst32 = edge_index[1].reshape(_NS * _NC, n32, _CH)
    zeros = jnp.zeros((_N, 128), jnp.float32)

    x2 = jnp.stack([features, cluster_features])
    for w, b, cout in ((conv_w0, conv_b0, 256), (conv_w1, conv_b1, 256),
                       (conv_w2, conv_b2, 128)):
        agg = _sc_segsum(x2.reshape(2 * _N, 128), src16, dst16, aff16, zeros,
                         split_channels=True)
        out = _tc_conv(x2, agg.reshape(2, _N, 128), w, b, cout)
        x2 = out
    x3 = x2
    aggp = _sc_segsum(x3, src32b, dst32b, aff32b, zeros, split_channels=False)
    sd = _tc_conv3(x3, aggp.reshape(2, _N, 128), conv_w3, conv_b3,
                   src_w, src_b, dst_w, dst_b)
    hsum = _sc_head(sd.reshape(2 * _N, 128), src32, dst32)
    pred_conn, m2 = _tc_head(hsum, raw_affine.reshape(_E // _BE, 1, _BE),
                             prelu1, cls_w1, cls_b1, prelu2, cls_w2, cls_b2)
    parts = _sc_den_partials(m2.reshape(_NS * _NC, n32 // _SB, _SB, _CH),
                             dst32b, zeros)
    pred_den = _tc_den(parts.reshape(2, _N, 128)).reshape(_N)
    return pred_conn, pred_den
